# Initial kernel scaffold; baseline (speedup 1.0000x reference)
import jax
import jax.numpy as jnp
from jax.experimental import pallas as pl


def kernel(movie_id, movie_categories, W_movies, W_cats, W_bias, fc_W, fc_b):
    raise NotImplementedError("write your pallas kernel here")


# speedup vs baseline: 2.9064x; 2.9064x over previous
import functools

import jax
import jax.numpy as jnp
from jax import lax
from jax.experimental import pallas as pl
from jax.experimental.pallas import tpu as pltpu
from jax.experimental.pallas import tpu_sc as plsc

B = 16384
L = 20
D_M = 32
D_C = 16
OUT = 20
NUM_CATS = 1000
NUM_MOVIES = 1000000

NC = 2
NS = 16
NW = NC * NS
CH = B // NW
IDX_MINOR = 128
NCHUNK = CH // IDX_MINOR
NG = CH // 16

GW = 128
MPACK = GW // D_M
WM_ROWS = NUM_MOVIES // MPACK
WB_ROWS = (NUM_MOVIES + GW - 1) // GW
WB_PAD = WB_ROWS * GW - NUM_MOVIES


def _sc_gather_bag(mid3, mcol2, bidx3, bcol2, rowids, cats16t, wm128, wb128,
                   wc_flat):
    mesh = plsc.VectorSubcoreMesh(core_axis_name="c", subcore_axis_name="s")

    @functools.partial(
        pl.kernel,
        mesh=mesh,
        compiler_params=pltpu.CompilerParams(
            needs_layout_passes=False, use_tc_tiling_on_sc=False),
        out_type=[
            jax.ShapeDtypeStruct((NW, D_M, CH), jnp.float32),
            jax.ShapeDtypeStruct((NW, D_C, CH), jnp.float32),
            jax.ShapeDtypeStruct((NW, CH), jnp.float32),
        ],
        scratch_types=[
            pltpu.VMEM((NCHUNK, IDX_MINOR), jnp.int32),
            pltpu.VMEM((NCHUNK, IDX_MINOR), jnp.int32),
            pltpu.VMEM((CH, GW), jnp.float32),
            pltpu.VMEM((D_M, CH), jnp.float32),
            pltpu.VMEM((NUM_CATS * D_C,), jnp.float32),
            pltpu.VMEM((L, CH), jnp.int32),
            pltpu.VMEM((D_C, CH), jnp.float32),
            pltpu.VMEM((CH,), jnp.int32),
            pltpu.VMEM((CH,), jnp.int32),
            pltpu.VMEM((CH,), jnp.int32),
            pltpu.VMEM((CH,), jnp.float32),
            pltpu.SemaphoreType.DMA,
            pltpu.SemaphoreType.DMA,
        ],
    )
    def body(mid_hbm, mcol_hbm, bidx_hbm, bcol_hbm, rowid_hbm, cats_hbm,
             wm_hbm, wb_hbm, wc_hbm,
             mrows_out, sums_out, bias_out,
             mid_v, bidx_v, stage_v, mrowsT_v, wc_v, cats_v, sumsT_v,
             mcol_v, bcol_v, rowid_v, bias_v, sem_m, sem_b):
        w = lax.axis_index("s") * NC + lax.axis_index("c")

        pltpu.sync_copy(mid_hbm.at[w], mid_v)
        mcopies = []
        for k in range(NCHUNK):
            sl = pl.ds(k * IDX_MINOR, IDX_MINOR)
            mcopies.append(pltpu.async_copy(
                wm_hbm.at[mid_v.at[k]], stage_v.at[sl], sem_m))

        pltpu.sync_copy(wc_hbm, wc_v)
        pltpu.sync_copy(cats_hbm.at[w], cats_v)
        pltpu.sync_copy(mcol_hbm.at[w], mcol_v)
        pltpu.sync_copy(rowid_hbm, rowid_v)
        pltpu.sync_copy(bidx_hbm.at[w], bidx_v)
        pltpu.sync_copy(bcol_hbm.at[w], bcol_v)

        def group_body(g, carry):
            s16 = pl.ds(g * 16, 16)
            cvecs = [cats_v[l, s16] for l in range(L)]
            for d in range(D_C):
                acc = plsc.load_gather(wc_v, [cvecs[0] + d])
                for l in range(1, L):
                    acc = acc + plsc.load_gather(wc_v, [cvecs[l] + d])
                sumsT_v[d, s16] = acc
            return carry

        lax.fori_loop(0, NG, group_body, 0)

        for cp in mcopies:
            cp.wait()

        def movie_body(g, carry):
            s16 = pl.ds(g * 16, 16)
            rvec = rowid_v[s16]
            cvec = mcol_v[s16]
            for d in range(D_M):
                mrowsT_v[d, s16] = plsc.load_gather(stage_v, [rvec, cvec + d])
            return carry

        lax.fori_loop(0, NG, movie_body, 0)
        pltpu.sync_copy(mrowsT_v, mrows_out.at[w])
        pltpu.sync_copy(sumsT_v, sums_out.at[w])

        bcopies = []
        for k in range(NCHUNK):
            sl = pl.ds(k * IDX_MINOR, IDX_MINOR)
            bcopies.append(pltpu.async_copy(
                wb_hbm.at[bidx_v.at[k]], stage_v.at[sl], sem_b))
        for cp in bcopies:
            cp.wait()

        def bias_body(g, carry):
            s16 = pl.ds(g * 16, 16)
            rvec = rowid_v[s16]
            cvec = bcol_v[s16]
            bias_v[s16] = plsc.load_gather(stage_v, [rvec, cvec])
            return carry

        lax.fori_loop(0, NG, bias_body, 0)
        pltpu.sync_copy(bias_v, bias_out.at[w])

    return body(mid3, mcol2, bidx3, bcol2, rowids, cats16t, wm128, wb128,
                wc_flat)


WPB = 4
BT = WPB * CH


def _tc_body(mrowsT_ref, sumsT_ref, cats_ref, fcm_ref, fcc_ref, fcb_ref,
             out_ref):
    cats = cats_ref[...]
    cnt = jnp.sum((cats != 0).astype(jnp.float32), axis=1)
    inv = jnp.where(cnt > 0.0, 1.0 / jnp.maximum(cnt, 1.0), 0.0)
    catvT = sumsT_ref[...] * inv[:, None, :]
    mparts = [
        lax.dot_general(mrowsT_ref[i], fcm_ref[...], (((0,), (0,)), ((), ())),
                        preferred_element_type=jnp.float32)
        for i in range(WPB)
    ]
    cparts = [
        lax.dot_general(catvT[i], fcc_ref[...], (((0,), (0,)), ((), ())),
                        preferred_element_type=jnp.float32)
        for i in range(WPB)
    ]
    mv = jnp.stack(mparts, axis=0)
    cv = jnp.stack(cparts, axis=0)
    out_ref[...] = mv + cv + fcb_ref[...].reshape(1, 1, OUT)


def _tc_head(mrowsT, sumsT, cats16t, fcm, fcc, fcb):
    return pl.pallas_call(
        _tc_body,
        grid=(NW // WPB,),
        in_specs=[
            pl.BlockSpec((WPB, D_M, CH), lambda i: (i, 0, 0)),
            pl.BlockSpec((WPB, D_C, CH), lambda i: (i, 0, 0)),
            pl.BlockSpec((WPB, L, CH), lambda i: (i, 0, 0)),
            pl.BlockSpec((D_M, OUT), lambda i: (0, 0)),
            pl.BlockSpec((D_C, OUT), lambda i: (0, 0)),
            pl.BlockSpec((1, OUT), lambda i: (0, 0)),
        ],
        out_specs=pl.BlockSpec((WPB, CH, OUT), lambda i: (i, 0, 0)),
        out_shape=jax.ShapeDtypeStruct((NW, CH, OUT), jnp.float32),
    )(mrowsT, sumsT, cats16t, fcm, fcc, fcb)


def kernel(movie_id, movie_categories, W_movies, W_cats, W_bias, fc_W, fc_b):
    mid = movie_id.astype(jnp.int32)
    cats_i32 = movie_categories.astype(jnp.int32)
    mid3 = (mid // MPACK).reshape(NW, NCHUNK, IDX_MINOR)
    mcol2 = ((mid % MPACK) * D_M).reshape(NW, CH)
    bidx3 = (mid // GW).reshape(NW, NCHUNK, IDX_MINOR)
    bcol2 = (mid % GW).reshape(NW, CH)
    rowids = jnp.arange(CH, dtype=jnp.int32)
    cats16t = (cats_i32 * D_C).reshape(NW, CH, L).transpose(0, 2, 1)
    wm128 = W_movies.reshape(WM_ROWS, GW)
    wb128 = jnp.concatenate(
        [W_bias.reshape(NUM_MOVIES),
         jnp.zeros((WB_PAD,), jnp.float32)]).reshape(WB_ROWS, GW)
    wc_flat = W_cats.reshape(NUM_CATS * D_C)
    mrowsT, sumsT, bias = _sc_gather_bag(
        mid3, mcol2, bidx3, bcol2, rowids, cats16t, wm128, wb128, wc_flat)
    fcm = fc_W[:, :D_M].T
    fcc = fc_W[:, D_M:].T
    fcb = fc_b.reshape(1, OUT)
    movie_vec = _tc_head(mrowsT, sumsT, cats16t, fcm, fcc, fcb)
    return movie_vec.reshape(B, OUT), bias.reshape(B)
